# baseline (speedup 1.0000x reference)
import functools

import jax
import jax.numpy as jnp
from jax import lax
from jax.experimental import pallas as pl
from jax.experimental.pallas import tpu as pltpu
from jax.experimental.pallas import tpu_sc as plsc

N0 = 10000
N1 = 160000
NF = 160000
D0 = 128
D1 = 128

N0P = 10240
NFP = 163840

ACH = 160
AGA = 40
ICH = 52

_SC_PARAMS = pltpu.CompilerParams(use_tc_tiling_on_sc=False)

_MESH = plsc.VectorSubcoreMesh(core_axis_name="c", subcore_axis_name="s")


def _x1_pass_body(x_ref, w_ref, z_ref, cs_ref):
    i = pl.program_id(0)
    x = x_ref[...]
    z_ref[...] = jnp.dot(x, w_ref[...], preferred_element_type=jnp.float32)

    @pl.when(i == 0)
    def _():
        cs_ref[...] = jnp.zeros_like(cs_ref)

    cs_ref[...] += jnp.sum(x, axis=0, keepdims=True)


def _x1_pass(x1, w12p):
    blk = 3200
    return pl.pallas_call(
        _x1_pass_body,
        grid=(N1 // blk,),
        in_specs=[
            pl.BlockSpec((blk, D1), lambda i: (i, 0)),
            pl.BlockSpec((D1, 8), lambda i: (0, 0)),
        ],
        out_specs=[
            pl.BlockSpec((blk, 8), lambda i: (i, 0)),
            pl.BlockSpec((1, D1), lambda i: (0, 0)),
        ],
        out_shape=[
            jax.ShapeDtypeStruct((N1, 8), jnp.float32),
            jax.ShapeDtypeStruct((1, D1), jnp.float32),
        ],
    )(x1, w12p)


def _mm_split_body(x_ref, w_ref, o_ref):
    h = jnp.dot(x_ref[...], w_ref[...], preferred_element_type=jnp.float32)
    o_ref[0] = h[:, :64]
    o_ref[1] = h[:, 64:]


def _mm_split(x0p, w):
    blk = 2048
    return pl.pallas_call(
        _mm_split_body,
        grid=(N0P // blk,),
        in_specs=[
            pl.BlockSpec((blk, D0), lambda i: (i, 0)),
            pl.BlockSpec((D0, D0), lambda i: (0, 0)),
        ],
        out_specs=pl.BlockSpec((2, blk, 64), lambda i: (0, i, 0)),
        out_shape=jax.ShapeDtypeStruct((2, N0P, 64), jnp.float32),
    )(x0p, w)


def _relu_mm_split_body(a_ref, w_ref, o_ref):
    x = jnp.concatenate(
        [jnp.maximum(a_ref[0], 0.0), jnp.maximum(a_ref[1], 0.0)], axis=1)
    h = jnp.dot(x, w_ref[...], preferred_element_type=jnp.float32)
    o_ref[0] = h[:, :64]
    o_ref[1] = h[:, 64:]


def _relu_mm_split(acc, w):
    blk = 2048
    return pl.pallas_call(
        _relu_mm_split_body,
        grid=(N0P // blk,),
        in_specs=[
            pl.BlockSpec((2, blk, 64), lambda i: (0, i, 0)),
            pl.BlockSpec((D0, D0), lambda i: (0, 0)),
        ],
        out_specs=pl.BlockSpec((2, blk, 64), lambda i: (0, i, 0)),
        out_shape=jax.ShapeDtypeStruct((2, N0P, 64), jnp.float32),
    )(acc, w)


def _final_body(f_ref, a_ref, cs_ref, l0w_ref, l0b_ref, l1w_ref, l1b_ref,
                l2w_ref, l2b_ref, o_ref, s0_ref, s2_ref):
    i = pl.program_id(0)
    blk = 2048

    @pl.when(i == 0)
    def _():
        s0_ref[...] = jnp.zeros_like(s0_ref)
        s2_ref[...] = jnp.zeros_like(s2_ref)

    @pl.when(i < 80)
    def _():
        rows = i * blk + lax.broadcasted_iota(jnp.int32, (blk, 1), 0)
        f = jnp.maximum(f_ref[0] + f_ref[1], 0.0)
        f = jnp.where(rows < NF, f, 0.0)
        s2_ref[...] += jnp.sum(f, axis=0, keepdims=True)

    @pl.when(i >= 80)
    def _():
        j = i - 80
        rows = j * blk + lax.broadcasted_iota(jnp.int32, (blk, 1), 0)
        keep = rows < N0
        lo = jnp.where(keep, jnp.maximum(a_ref[0], 0.0), 0.0)
        hi = jnp.where(keep, jnp.maximum(a_ref[1], 0.0), 0.0)
        s0_ref[0, :64] += jnp.sum(lo, axis=0)
        s0_ref[0, 64:] += jnp.sum(hi, axis=0)

    @pl.when(i == 84)
    def _():
        m0 = s0_ref[...] * (1.0 / N0)
        m1 = cs_ref[...] * (1.0 / N1)
        m2 = s2_ref[...] * (1.0 / NF)
        o_ref[...] = (
            jnp.dot(m0, l0w_ref[...], preferred_element_type=jnp.float32)
            + l0b_ref[...]
            + jnp.dot(m1, l1w_ref[...], preferred_element_type=jnp.float32)
            + l1b_ref[...]
            + jnp.dot(m2, l2w_ref[...], preferred_element_type=jnp.float32)
            + l2b_ref[...]
        )


def _final_reduce(facc, acc1, cs1, l0w, l0b, l1w, l1b, l2wp, l2b):
    blk = 2048
    return pl.pallas_call(
        _final_body,
        grid=(85,),
        in_specs=[
            pl.BlockSpec((2, blk, 8), lambda i: (0, jnp.minimum(i, 79), 0)),
            pl.BlockSpec((2, blk, 64), lambda i: (0, jnp.maximum(i - 80, 0), 0)),
            pl.BlockSpec((1, D1), lambda i: (0, 0)),
            pl.BlockSpec((D0, 2), lambda i: (0, 0)),
            pl.BlockSpec((1, 2), lambda i: (0, 0)),
            pl.BlockSpec((D1, 2), lambda i: (0, 0)),
            pl.BlockSpec((1, 2), lambda i: (0, 0)),
            pl.BlockSpec((8, 2), lambda i: (0, 0)),
            pl.BlockSpec((1, 2), lambda i: (0, 0)),
        ],
        out_specs=pl.BlockSpec((1, 2), lambda i: (0, 0)),
        out_shape=jax.ShapeDtypeStruct((1, 2), jnp.float32),
        scratch_shapes=[
            pltpu.VMEM((1, D0), jnp.float32),
            pltpu.VMEM((1, 8), jnp.float32),
        ],
    )(facc, acc1, cs1, l0w, l0b, l1w, l1b, l2wp, l2b)


@functools.partial(
    pl.kernel,
    out_type=jax.ShapeDtypeStruct((2, N0P, 64), jnp.float32),
    mesh=_MESH,
    scratch_types=[
        pltpu.VMEM_SHARED((N0P, 64), jnp.float32),
        pltpu.VMEM_SHARED((N0P, 64), jnp.float32),
        pltpu.VMEM((AGA, 2, 128), jnp.int32),
        [pltpu.VMEM((128, 64), jnp.float32)] * 4,
        [pltpu.SemaphoreType.DMA] * 4,
        [pltpu.SemaphoreType.DMA] * 4,
    ],
    compiler_params=_SC_PARAMS,
)
def _adj_spmm(h_hbm, idx_hbm, z_hbm, out_hbm,
              h_sh, acc_sh, idx_v, gbufs, gsems, ssems):
    c = lax.axis_index("c")
    s = lax.axis_index("s")
    stripe = pl.ds(s * (N0P // 16), N0P // 16)
    pltpu.sync_copy(h_hbm.at[c, stripe], h_sh.at[stripe])
    pltpu.sync_copy(z_hbm, acc_sh.at[stripe])
    plsc.subcore_barrier()

    def _drain_scatter(b):
        pltpu.make_async_copy(
            h_hbm.at[0, pl.ds(0, 128)], gbufs[b], ssems[b]).wait()

    def quad(q, carry2):
        gd = []
        for b in range(4):
            @pl.when(q > 0)
            def _():
                _drain_scatter(b)
            gd.append(pltpu.async_copy(
                h_sh.at[idx_v.at[4 * q + b, 0]], gbufs[b], gsems[b]))
        for b in range(4):
            gd[b].wait()
            pltpu.async_copy(
                gbufs[b], acc_sh.at[idx_v.at[4 * q + b, 1]], ssems[b],
                add=True)
        return carry2

    def group_loop(g, carry):
        @pl.when(g > 0)
        def _():
            for b in range(4):
                _drain_scatter(b)
        pltpu.sync_copy(idx_hbm.at[s, pl.ds(g * AGA, AGA)], idx_v)
        lax.fori_loop(0, AGA // 4, quad, 0)
        return carry

    lax.fori_loop(0, ACH // AGA, group_loop, 0)
    for b in range(4):
        _drain_scatter(b)
    plsc.subcore_barrier()
    pltpu.sync_copy(acc_sh.at[stripe], out_hbm.at[c, stripe])


@functools.partial(
    pl.kernel,
    out_type=jax.ShapeDtypeStruct((2, NFP, 8), jnp.float32),
    mesh=_MESH,
    scratch_types=[
        pltpu.VMEM_SHARED((NFP, 8), jnp.float32),
        pltpu.VMEM((ICH, 2, 128), jnp.int32),
        [pltpu.VMEM((128, 8), jnp.float32)] * 4,
        [pltpu.SemaphoreType.DMA] * 4,
        [pltpu.SemaphoreType.DMA] * 4,
    ],
    compiler_params=_SC_PARAMS,
)
def _inc_spmm(z1_hbm, idx_hbm, z_hbm, out_hbm,
              acc_sh, idx_v, gbufs, gsems, ssems):
    c = lax.axis_index("c")
    s = lax.axis_index("s")
    stripe = pl.ds(s * (NFP // 16), NFP // 16)
    pltpu.sync_copy(z_hbm, acc_sh.at[stripe])
    pltpu.sync_copy(idx_hbm.at[c, s], idx_v)
    plsc.subcore_barrier()

    def quad(q, carry):
        gd = []
        for b in range(4):
            @pl.when(q > 0)
            def _():
                pltpu.make_async_copy(
                    z1_hbm.at[pl.ds(0, 128)], gbufs[b], ssems[b]).wait()
            gd.append(pltpu.async_copy(
                z1_hbm.at[idx_v.at[4 * q + b, 0]], gbufs[b], gsems[b]))
        for b in range(4):
            gd[b].wait()
            pltpu.async_copy(
                gbufs[b], acc_sh.at[idx_v.at[4 * q + b, 1]], ssems[b],
                add=True)
        return carry

    lax.fori_loop(0, ICH // 4, quad, 0)
    for b in range(4):
        pltpu.make_async_copy(
            z1_hbm.at[pl.ds(0, 128)], gbufs[b], ssems[b]).wait()
    plsc.subcore_barrier()
    pltpu.sync_copy(acc_sh.at[stripe], out_hbm.at[c, stripe])


def kernel(x_0, x_1, adjacency_0, incidence_2_t,
           W0_0, W12_0, W0_1, W12_1,
           lin0_w, lin0_b, lin1_w, lin1_b, lin2_w, lin2_b):
    f32 = jnp.float32
    i32 = jnp.int32

    x0p = jnp.zeros((N0P, D0), f32).at[:N0].set(x_0.astype(f32))
    w12p = jnp.zeros((D1, 8), f32).at[:, :5].set(W12_1.astype(f32))
    l2wp = jnp.zeros((8, 2), f32).at[:5].set(lin2_w.astype(f32))

    ea = 16 * ACH * 128
    arow = jnp.concatenate(
        [adjacency_0[0].astype(i32),
         jnp.full((ea - adjacency_0.shape[1],), N0, i32)]).reshape(16, ACH, 128)
    acol = jnp.concatenate(
        [adjacency_0[1].astype(i32),
         jnp.zeros((ea - adjacency_0.shape[1],), i32)]).reshape(16, ACH, 128)
    aidx = jnp.stack([acol, arow], axis=2)

    ei = 2 * 16 * ICH * 128
    irow = jnp.concatenate(
        [incidence_2_t[0].astype(i32),
         jnp.full((ei - incidence_2_t.shape[1],), NF, i32)]
    ).reshape(2, 16, ICH, 128)
    icol = jnp.concatenate(
        [incidence_2_t[1].astype(i32),
         jnp.zeros((ei - incidence_2_t.shape[1],), i32)]
    ).reshape(2, 16, ICH, 128)
    iidx = jnp.stack([icol, irow], axis=3)

    zeros_a = jnp.zeros((N0P // 16, 64), f32)
    zeros_i = jnp.zeros((NFP // 16, 8), f32)

    z1, cs1 = _x1_pass(x_1.astype(f32), w12p)
    h0 = _mm_split(x0p, W0_0.astype(f32))
    a0 = _adj_spmm(h0, aidx, zeros_a)
    h1 = _relu_mm_split(a0, W0_1.astype(f32))
    a1 = _adj_spmm(h1, aidx, zeros_a)
    facc = _inc_spmm(z1, iidx, zeros_i)

    out = _final_reduce(
        facc, a1, cs1,
        lin0_w.astype(f32), lin0_b.astype(f32).reshape(1, 2),
        lin1_w.astype(f32), lin1_b.astype(f32).reshape(1, 2),
        l2wp, lin2_b.astype(f32).reshape(1, 2))
    return out.reshape(2)


# speedup vs baseline: 1.0669x; 1.0669x over previous
import functools

import jax
import jax.numpy as jnp
from jax import lax
from jax.experimental import pallas as pl
from jax.experimental.pallas import tpu as pltpu
from jax.experimental.pallas import tpu_sc as plsc

N0 = 10000
N1 = 160000
NF = 160000
D0 = 128
D1 = 128

N0P = 10240
NFP = 163840

ACH = 160
AGA = 40
ICH = 52

_SC_PARAMS = pltpu.CompilerParams(use_tc_tiling_on_sc=False)

_MESH = plsc.VectorSubcoreMesh(core_axis_name="c", subcore_axis_name="s")


def _x1_pass_body(x_ref, w_ref, z_ref, cs_ref):
    i = pl.program_id(0)
    x = x_ref[...]
    z_ref[...] = jnp.dot(x, w_ref[...], preferred_element_type=jnp.float32)

    @pl.when(i == 0)
    def _():
        cs_ref[...] = jnp.zeros_like(cs_ref)

    cs_ref[...] += jnp.sum(x, axis=0, keepdims=True)


def _x1_pass(x1, w12p):
    blk = 3200
    return pl.pallas_call(
        _x1_pass_body,
        grid=(N1 // blk,),
        in_specs=[
            pl.BlockSpec((blk, D1), lambda i: (i, 0)),
            pl.BlockSpec((D1, 8), lambda i: (0, 0)),
        ],
        out_specs=[
            pl.BlockSpec((blk, 8), lambda i: (i, 0)),
            pl.BlockSpec((1, D1), lambda i: (0, 0)),
        ],
        out_shape=[
            jax.ShapeDtypeStruct((N1, 8), jnp.float32),
            jax.ShapeDtypeStruct((1, D1), jnp.float32),
        ],
    )(x1, w12p)


def _mm_split_body(x_ref, w_ref, o_ref):
    h = jnp.dot(x_ref[...], w_ref[...], preferred_element_type=jnp.float32)
    o_ref[0] = h[:, :64]
    o_ref[1] = h[:, 64:]


def _mm_split(x0p, w):
    blk = 2048
    return pl.pallas_call(
        _mm_split_body,
        grid=(N0P // blk,),
        in_specs=[
            pl.BlockSpec((blk, D0), lambda i: (i, 0)),
            pl.BlockSpec((D0, D0), lambda i: (0, 0)),
        ],
        out_specs=pl.BlockSpec((2, blk, 64), lambda i: (0, i, 0)),
        out_shape=jax.ShapeDtypeStruct((2, N0P, 64), jnp.float32),
    )(x0p, w)


def _relu_mm_split_body(a_ref, w_ref, o_ref):
    x = jnp.concatenate(
        [jnp.maximum(a_ref[0], 0.0), jnp.maximum(a_ref[1], 0.0)], axis=1)
    h = jnp.dot(x, w_ref[...], preferred_element_type=jnp.float32)
    o_ref[0] = h[:, :64]
    o_ref[1] = h[:, 64:]


def _relu_mm_split(acc, w):
    blk = 2048
    return pl.pallas_call(
        _relu_mm_split_body,
        grid=(N0P // blk,),
        in_specs=[
            pl.BlockSpec((2, blk, 64), lambda i: (0, i, 0)),
            pl.BlockSpec((D0, D0), lambda i: (0, 0)),
        ],
        out_specs=pl.BlockSpec((2, blk, 64), lambda i: (0, i, 0)),
        out_shape=jax.ShapeDtypeStruct((2, N0P, 64), jnp.float32),
    )(acc, w)


def _final_body(f_ref, a_ref, cs_ref, l0w_ref, l0b_ref, l1w_ref, l1b_ref,
                l2w_ref, l2b_ref, o_ref, s0_ref, s2_ref):
    i = pl.program_id(0)
    blk = 2048

    @pl.when(i == 0)
    def _():
        s0_ref[...] = jnp.zeros_like(s0_ref)
        s2_ref[...] = jnp.zeros_like(s2_ref)

    @pl.when(i < 80)
    def _():
        rows = i * blk + lax.broadcasted_iota(jnp.int32, (blk, 1), 0)
        f = jnp.maximum(f_ref[0] + f_ref[1], 0.0)
        f = jnp.where(rows < NF, f, 0.0)
        s2_ref[...] += jnp.sum(f, axis=0, keepdims=True)

    @pl.when(i >= 80)
    def _():
        j = i - 80
        rows = j * blk + lax.broadcasted_iota(jnp.int32, (blk, 1), 0)
        keep = rows < N0
        lo = jnp.where(keep, jnp.maximum(a_ref[0], 0.0), 0.0)
        hi = jnp.where(keep, jnp.maximum(a_ref[1], 0.0), 0.0)
        s0_ref[0, :64] += jnp.sum(lo, axis=0)
        s0_ref[0, 64:] += jnp.sum(hi, axis=0)

    @pl.when(i == 84)
    def _():
        m0 = s0_ref[...] * (1.0 / N0)
        m1 = cs_ref[...] * (1.0 / N1)
        m2 = s2_ref[...] * (1.0 / NF)
        o_ref[...] = (
            jnp.dot(m0, l0w_ref[...], preferred_element_type=jnp.float32)
            + l0b_ref[...]
            + jnp.dot(m1, l1w_ref[...], preferred_element_type=jnp.float32)
            + l1b_ref[...]
            + jnp.dot(m2, l2w_ref[...], preferred_element_type=jnp.float32)
            + l2b_ref[...]
        )


def _final_reduce(facc, acc1, cs1, l0w, l0b, l1w, l1b, l2wp, l2b):
    blk = 2048
    return pl.pallas_call(
        _final_body,
        grid=(85,),
        in_specs=[
            pl.BlockSpec((2, blk, 8), lambda i: (0, jnp.minimum(i, 79), 0)),
            pl.BlockSpec((2, blk, 64), lambda i: (0, jnp.maximum(i - 80, 0), 0)),
            pl.BlockSpec((1, D1), lambda i: (0, 0)),
            pl.BlockSpec((D0, 2), lambda i: (0, 0)),
            pl.BlockSpec((1, 2), lambda i: (0, 0)),
            pl.BlockSpec((D1, 2), lambda i: (0, 0)),
            pl.BlockSpec((1, 2), lambda i: (0, 0)),
            pl.BlockSpec((8, 2), lambda i: (0, 0)),
            pl.BlockSpec((1, 2), lambda i: (0, 0)),
        ],
        out_specs=pl.BlockSpec((1, 2), lambda i: (0, 0)),
        out_shape=jax.ShapeDtypeStruct((1, 2), jnp.float32),
        scratch_shapes=[
            pltpu.VMEM((1, D0), jnp.float32),
            pltpu.VMEM((1, 8), jnp.float32),
        ],
    )(facc, acc1, cs1, l0w, l0b, l1w, l1b, l2wp, l2b)


@functools.partial(
    pl.kernel,
    out_type=jax.ShapeDtypeStruct((2, N0P, 64), jnp.float32),
    mesh=_MESH,
    scratch_types=[
        pltpu.VMEM_SHARED((N0P, 64), jnp.float32),
        pltpu.VMEM_SHARED((N0P, 64), jnp.float32),
        pltpu.VMEM((AGA, 2, 128), jnp.int32),
        [pltpu.VMEM((128, 64), jnp.float32)] * 4,
        [pltpu.SemaphoreType.DMA] * 4,
        [pltpu.SemaphoreType.DMA] * 4,
    ],
    compiler_params=_SC_PARAMS,
)
def _adj_spmm(h_hbm, idx_hbm, z_hbm, out_hbm,
              h_sh, acc_sh, idx_v, gbufs, gsems, ssems):
    c = lax.axis_index("c")
    s = lax.axis_index("s")
    stripe = pl.ds(s * (N0P // 16), N0P // 16)
    pltpu.sync_copy(h_hbm.at[c, stripe], h_sh.at[stripe])
    pltpu.sync_copy(z_hbm, acc_sh.at[stripe])
    plsc.subcore_barrier()

    def quad(q, carry2):
        gd = [
            pltpu.async_copy(
                h_sh.at[idx_v.at[4 * q + b, 0]], gbufs[b], gsems[b])
            for b in range(4)
        ]
        sd = []
        for b in range(4):
            gd[b].wait()
            sd.append(pltpu.async_copy(
                gbufs[b], acc_sh.at[idx_v.at[4 * q + b, 1]], ssems[b],
                add=True))
        for d in sd:
            d.wait()
        return carry2

    def group_loop(g, carry):
        pltpu.sync_copy(idx_hbm.at[s, pl.ds(g * AGA, AGA)], idx_v)
        lax.fori_loop(0, AGA // 4, quad, 0)
        return carry

    lax.fori_loop(0, ACH // AGA, group_loop, 0)
    plsc.subcore_barrier()
    pltpu.sync_copy(acc_sh.at[stripe], out_hbm.at[c, stripe])


@functools.partial(
    pl.kernel,
    out_type=jax.ShapeDtypeStruct((2, NFP, 8), jnp.float32),
    mesh=_MESH,
    scratch_types=[
        pltpu.VMEM_SHARED((NFP, 8), jnp.float32),
        pltpu.VMEM((ICH, 2, 128), jnp.int32),
        [pltpu.VMEM((128, 8), jnp.float32)] * 4,
        [pltpu.SemaphoreType.DMA] * 4,
        [pltpu.SemaphoreType.DMA] * 4,
    ],
    compiler_params=_SC_PARAMS,
)
def _inc_spmm(z1_hbm, idx_hbm, z_hbm, out_hbm,
              acc_sh, idx_v, gbufs, gsems, ssems):
    c = lax.axis_index("c")
    s = lax.axis_index("s")
    stripe = pl.ds(s * (NFP // 16), NFP // 16)
    pltpu.sync_copy(z_hbm, acc_sh.at[stripe])
    pltpu.sync_copy(idx_hbm.at[c, s], idx_v)
    plsc.subcore_barrier()

    def quad(q, carry):
        gd = []
        for b in range(4):
            @pl.when(q > 0)
            def _():
                pltpu.make_async_copy(
                    z1_hbm.at[pl.ds(0, 128)], gbufs[b], ssems[b]).wait()
            gd.append(pltpu.async_copy(
                z1_hbm.at[idx_v.at[4 * q + b, 0]], gbufs[b], gsems[b]))
        for b in range(4):
            gd[b].wait()
            pltpu.async_copy(
                gbufs[b], acc_sh.at[idx_v.at[4 * q + b, 1]], ssems[b],
                add=True)
        return carry

    lax.fori_loop(0, ICH // 4, quad, 0)
    for b in range(4):
        pltpu.make_async_copy(
            z1_hbm.at[pl.ds(0, 128)], gbufs[b], ssems[b]).wait()
    plsc.subcore_barrier()
    pltpu.sync_copy(acc_sh.at[stripe], out_hbm.at[c, stripe])


def kernel(x_0, x_1, adjacency_0, incidence_2_t,
           W0_0, W12_0, W0_1, W12_1,
           lin0_w, lin0_b, lin1_w, lin1_b, lin2_w, lin2_b):
    f32 = jnp.float32
    i32 = jnp.int32

    x0p = jnp.zeros((N0P, D0), f32).at[:N0].set(x_0.astype(f32))
    w12p = jnp.zeros((D1, 8), f32).at[:, :5].set(W12_1.astype(f32))
    l2wp = jnp.zeros((8, 2), f32).at[:5].set(lin2_w.astype(f32))

    ea = 16 * ACH * 128
    arow = jnp.concatenate(
        [adjacency_0[0].astype(i32),
         jnp.full((ea - adjacency_0.shape[1],), N0, i32)]).reshape(16, ACH, 128)
    acol = jnp.concatenate(
        [adjacency_0[1].astype(i32),
         jnp.zeros((ea - adjacency_0.shape[1],), i32)]).reshape(16, ACH, 128)
    aidx = jnp.stack([acol, arow], axis=2)

    ei = 2 * 16 * ICH * 128
    irow = jnp.concatenate(
        [incidence_2_t[0].astype(i32),
         jnp.full((ei - incidence_2_t.shape[1],), NF, i32)]
    ).reshape(2, 16, ICH, 128)
    icol = jnp.concatenate(
        [incidence_2_t[1].astype(i32),
         jnp.zeros((ei - incidence_2_t.shape[1],), i32)]
    ).reshape(2, 16, ICH, 128)
    iidx = jnp.stack([icol, irow], axis=3)

    zeros_a = jnp.zeros((N0P // 16, 64), f32)
    zeros_i = jnp.zeros((NFP // 16, 8), f32)

    z1, cs1 = _x1_pass(x_1.astype(f32), w12p)
    h0 = _mm_split(x0p, W0_0.astype(f32))
    a0 = _adj_spmm(h0, aidx, zeros_a)
    h1 = _relu_mm_split(a0, W0_1.astype(f32))
    a1 = _adj_spmm(h1, aidx, zeros_a)
    facc = _inc_spmm(z1, iidx, zeros_i)

    out = _final_reduce(
        facc, a1, cs1,
        lin0_w.astype(f32), lin0_b.astype(f32).reshape(1, 2),
        lin1_w.astype(f32), lin1_b.astype(f32).reshape(1, 2),
        l2wp, lin2_b.astype(f32).reshape(1, 2))
    return out.reshape(2)
